# baseline (speedup 1.0000x reference)
import functools

import jax
import jax.numpy as jnp
from jax import lax
from jax.experimental import pallas as pl
from jax.experimental.pallas import tpu as pltpu
from jax.experimental.pallas import tpu_sc as plsc

GRID_RES = 128
GRID_DIM = 8
NUM_EMBEDDINGS = 12000
EMBEDDING_DIM = 8
TABLE_SIZE_ = NUM_EMBEDDINGS * GRID_DIM ** 3
RES_ = GRID_RES * GRID_DIM
HI_ = float(RES_ - 1 - 1e-5)
SCALE_ = 0.5 * (RES_ - 1)
K2_ = 2654435761
K3_ = 805459861

NC = 2
NS = 16
NW = NC * NS
L = 16

C = 512
NR = 8 * C
NSTREAM = NR // 128

CORNERS = [(dx, dy, dz) for dx in (0, 1) for dy in (0, 1) for dz in (0, 1)]

BISECT = 0

Bt = 4096


K2S_ = K2_ - 2 ** 32
K3S_ = K3_
TWO32_ = float(2 ** 32)
RCP_T_ = 1.0 / TABLE_SIZE_


def _vfull(val, dtype=jnp.float32):
    return jnp.full((16,), val, dtype)


def _dims(v):
    p_raw = (v + 1.0) * 0.5 * float(RES_ - 1)
    p = jnp.minimum(jnp.maximum(p_raw, 0.0), jnp.float32(HI_))
    i0 = p.astype(jnp.int32)
    fr = p - i0.astype(jnp.float32)
    i1 = jnp.minimum(i0 + 1, RES_ - 1)
    s = jnp.where((p_raw >= 0.0) & (p_raw <= jnp.float32(HI_)),
                  _vfull(SCALE_), _vfull(0.0))
    return i0, i1, fr, s


def _hash_mod(hx, hy, hz):
    h = hx ^ (hy * jnp.int32(K2S_)) ^ (hz * jnp.int32(K3S_))
    uh = h.astype(jnp.float32) + jnp.where(h < 0, _vfull(TWO32_), _vfull(0.0))
    q = (uh * jnp.float32(RCP_T_)).astype(jnp.int32)
    r = h - q * jnp.int32(TABLE_SIZE_)
    r = r + jnp.where(r < 0, _vfull(TABLE_SIZE_, jnp.int32),
                      _vfull(0, jnp.int32))
    r = r - jnp.where(r >= TABLE_SIZE_, _vfull(TABLE_SIZE_, jnp.int32),
                      _vfull(0, jnp.int32))
    return r


BPT = (TABLE_SIZE_ // 128) // NW
NBLK = 15


def _sc_transpose_body(tin, tout, in_v, out_v):
    cid = lax.axis_index("c")
    sid = lax.axis_index("s")
    wid = sid * NC + cid
    lanes = lax.iota(jnp.int32, 16)

    @pl.loop(0, BPT // NBLK)
    def _ch(k):
        b0 = wid * BPT + k * NBLK
        pltpu.sync_copy(tin.at[pl.ds(b0 * 1024, NBLK * 1024)], in_v)

        evec = jnp.where(lanes < 8, 2 * lanes, 2 * lanes - 15)
        dvecs = [((lanes & 7) + i) & 7 for i in range(8)]

        @pl.loop(0, NBLK * 8)
        def _g(g):
            eb = g // 8
            o = (g % 8) * 16
            ibase = eb * 1024 + o + evec
            obase = (eb * 128 + o + evec) * 8
            for i in range(8):
                t = plsc.load_gather(in_v, [ibase + dvecs[i] * 128])
                plsc.store_scatter(out_v, [obase + dvecs[i]], t)

        pltpu.sync_copy(out_v, tout.at[pl.ds(b0 * 1024, NBLK * 1024)])


def _sc_body(x_hbm, y_hbm, z_hbm, table_hbm, feats_hbm, mask_hbm,
             pos_v, idx_v, rows_v, out_v, mask_v, gsem, *, k_chunks, n_pad):
    cid = lax.axis_index("c")
    sid = lax.axis_index("s")
    wid = sid * NC + cid
    lanes = lax.iota(jnp.int32, 16)

    @pl.loop(0, k_chunks)
    def _chunk(k):
        base = (wid * k_chunks + k) * C
        pltpu.sync_copy(x_hbm.at[pl.ds(base, C)], pos_v.at[pl.ds(0, C)])
        pltpu.sync_copy(y_hbm.at[pl.ds(base, C)], pos_v.at[pl.ds(C, C)])
        pltpu.sync_copy(z_hbm.at[pl.ds(base, C)], pos_v.at[pl.ds(2 * C, C)])

        def _xyz(o):
            x = pos_v[pl.ds(o, 16)]
            y = pos_v[pl.ds(C + o, 16)]
            z = pos_v[pl.ds(2 * C + o, 16)]
            return x, y, z

        @pl.loop(0, C // 16)
        def _grp1(g):
            o = g * 16
            x, y, z = _xyz(o)
            ix0, ix1, _, _ = _dims(x)
            iy0, iy1, _, _ = _dims(y)
            iz0, iz1, _, _ = _dims(z)
            for c, (dx, dy, dz) in enumerate(CORNERS):
                h = _hash_mod((ix0, ix1)[dx], (iy0, iy1)[dy], (iz0, iz1)[dz])
                idx_v[pl.ds(c * C + o, 16)] = h
            m = ((jnp.abs(x) <= 1.0) & (jnp.abs(y) <= 1.0)
                 & (jnp.abs(z) <= 1.0))
            mask_v[pl.ds(o, 16)] = jnp.where(
                m, _vfull(1, jnp.int32), _vfull(0, jnp.int32))

        pltpu.async_copy(table_hbm.at[idx_v], rows_v, gsem).wait()

        if BISECT:
            return

        evec = jnp.where(lanes < 8, 2 * lanes, 2 * lanes - 15)
        dvecs = [((lanes & 7) + i) & 7 for i in range(8)]
        cvecs = [dvecs[i] * 128 + evec for i in range(8)]

        @pl.loop(0, C // 16)
        def _grp2(g):
            o = g * 16
            x = plsc.load_gather(pos_v, [o + evec])
            y = plsc.load_gather(pos_v, [C + o + evec])
            z = plsc.load_gather(pos_v, [2 * C + o + evec])
            _, _, fx_, sx = _dims(x)
            _, _, fy_, sy = _dims(y)
            _, _, fz_, sz = _dims(z)
            wx = (1.0 - fx_, fx_)
            wy = (1.0 - fy_, fy_)
            wz = (1.0 - fz_, fz_)
            wyz = {(a, b): wy[a] * wz[b] for a in (0, 1) for b in (0, 1)}
            wxz = {(a, b): wx[a] * wz[b] for a in (0, 1) for b in (0, 1)}
            wxy = {(a, b): wx[a] * wy[b] for a in (0, 1) for b in (0, 1)}
            uyz = {kk: v * sx for kk, v in wyz.items()}
            uxz = {kk: v * sy for kk, v in wxz.items()}
            uxy = {kk: v * sz for kk, v in wxy.items()}
            zero = jnp.zeros((16,), jnp.float32)
            accF = [zero] * 8
            accX = [zero] * 8
            accY = [zero] * 8
            accZ = [zero] * 8
            for c, (dx, dy, dz) in enumerate(CORNERS):
                m_c = wx[dx] * wyz[(dy, dz)]
                ax = uyz[(dy, dz)]
                ay = uxz[(dx, dz)]
                az = uxy[(dx, dy)]
                rbase = c * C + o + evec
                for i in range(8):
                    t = plsc.load_gather(rows_v, [rbase, dvecs[i]])
                    accF[i] = accF[i] + m_c * t
                    accX[i] = accX[i] + ax * t if dx else accX[i] - ax * t
                    accY[i] = accY[i] + ay * t if dy else accY[i] - ay * t
                    accZ[i] = accZ[i] + az * t if dz else accZ[i] - az * t
            blk = (o // 128) * (32 * 128) + (o % 128)
            for i in range(8):
                plsc.store_scatter(out_v, [blk + cvecs[i]], accF[i])
                plsc.store_scatter(out_v, [blk + 1024 + cvecs[i]], accX[i])
                plsc.store_scatter(out_v, [blk + 2048 + cvecs[i]], accY[i])
                plsc.store_scatter(out_v, [blk + 3072 + cvecs[i]], accZ[i])

        pltpu.sync_copy(out_v, feats_hbm.at[pl.ds(base * 32, C * 32)])
        pltpu.sync_copy(mask_v, mask_hbm.at[pl.ds(base, C)])


def _tc_body(feats_ref, w0t_ref, w1t_ref, w1_ref, w0_ref, w2_ref, w2t_ref,
             bias_ref, out_ref):
    f32 = jnp.float32
    gb = feats_ref.shape[0]
    big = jnp.concatenate([feats_ref[g] for g in range(gb)], axis=1)
    F = big[0:8, :]
    FX = big[8:16, :]
    FY = big[16:24, :]
    FZ = big[24:32, :]
    b0 = bias_ref[:, 0:1]
    b1 = bias_ref[:, 1:2]
    b2 = bias_ref[0:1, 2:3]
    A0 = jnp.dot(w0t_ref[...], F, preferred_element_type=f32) + b0
    h0 = jnp.maximum(A0, 0.0)
    A1 = jnp.dot(w1t_ref[...], h0, preferred_element_type=f32) + b1
    h1 = jnp.maximum(A1, 0.0)
    sdf = jnp.dot(w2t_ref[...], h1, preferred_element_type=f32) + b2
    g1 = jnp.where(A1 > 0.0, w2_ref[...], 0.0)
    g0 = jnp.where(A0 > 0.0,
                   jnp.dot(w1_ref[...], g1, preferred_element_type=f32), 0.0)
    gF = jnp.dot(w0_ref[...], g0, preferred_element_type=f32)
    gx = jnp.sum(gF * FX, axis=0, keepdims=True)
    gy = jnp.sum(gF * FY, axis=0, keepdims=True)
    gz = jnp.sum(gF * FZ, axis=0, keepdims=True)
    out_ref[0:4, :] = jnp.concatenate([sdf, gx, gy, gz], axis=0)


def kernel(positions, table, W0, b0, W1, b1, W2, b2):
    n = positions.shape[0]
    span = NW * C
    k_chunks = -(-n // span)
    n_pad = k_chunks * span

    xp = jnp.pad(positions[:, 0], (0, n_pad - n))
    yp = jnp.pad(positions[:, 1], (0, n_pad - n))
    zp = jnp.pad(positions[:, 2], (0, n_pad - n))

    mesh = plsc.VectorSubcoreMesh(core_axis_name="c", subcore_axis_name="s",
                                  num_cores=NC, num_subcores=NS)
    sc_params = pltpu.CompilerParams(use_tc_tiling_on_sc=False,
                                     needs_layout_passes=False)

    tiled_flat = table.reshape(TABLE_SIZE_ // 128, 128, 8)
    tiled_flat = tiled_flat.swapaxes(1, 2).reshape(-1)
    tr_fn = pl.kernel(
        _sc_transpose_body,
        out_type=jax.ShapeDtypeStruct((TABLE_SIZE_ * 8,), jnp.float32),
        mesh=mesh,
        scratch_types=[
            pltpu.VMEM((NBLK * 1024,), jnp.float32),
            pltpu.VMEM((NBLK * 1024,), jnp.float32),
        ],
        compiler_params=sc_params,
    )
    table_lin = tr_fn(tiled_flat).reshape(TABLE_SIZE_, 8)

    sc_fn = pl.kernel(
        functools.partial(_sc_body, k_chunks=k_chunks, n_pad=n_pad),
        out_type=(jax.ShapeDtypeStruct((32 * n_pad,), jnp.float32),
                  jax.ShapeDtypeStruct((n_pad,), jnp.int32)),
        mesh=mesh,
        scratch_types=[
            pltpu.VMEM((3 * C,), jnp.float32),
            pltpu.VMEM((NR,), jnp.int32),
            pltpu.VMEM((NR, 8), jnp.float32),
            pltpu.VMEM((32 * C,), jnp.float32),
            pltpu.VMEM((C,), jnp.int32),
            pltpu.SemaphoreType.DMA,
        ],
        compiler_params=sc_params,
    )
    feats, mask_i = sc_fn(xp, yp, zp, table_lin)

    bias_pack = jnp.zeros((32, 128), jnp.float32)
    bias_pack = bias_pack.at[:, 0].set(b0)
    bias_pack = bias_pack.at[:, 1].set(b1)
    bias_pack = bias_pack.at[0, 2].set(b2[0])

    feats3 = feats.reshape(n_pad // 128, 32, 128)
    grid = n_pad // Bt
    out4 = pl.pallas_call(
        _tc_body,
        grid=(grid,),
        in_specs=[
            pl.BlockSpec((Bt // 128, 32, 128), lambda i: (i, 0, 0)),
            pl.BlockSpec((32, 8), lambda i: (0, 0)),
            pl.BlockSpec((32, 32), lambda i: (0, 0)),
            pl.BlockSpec((32, 32), lambda i: (0, 0)),
            pl.BlockSpec((8, 32), lambda i: (0, 0)),
            pl.BlockSpec((32, 1), lambda i: (0, 0)),
            pl.BlockSpec((1, 32), lambda i: (0, 0)),
            pl.BlockSpec((32, 128), lambda i: (0, 0)),
        ],
        out_specs=pl.BlockSpec((8, Bt), lambda i: (0, i)),
        out_shape=jax.ShapeDtypeStruct((8, n_pad), jnp.float32),
    )(feats3, W0.T, W1.T, W1, W0, W2, W2.T, bias_pack)

    sdf = out4[0, :n].reshape(n, 1)
    grad_x = out4[1:4, :n].T
    mask = mask_i[:n].astype(bool)
    return (sdf, grad_x, mask)


# speedup vs baseline: 1.0293x; 1.0293x over previous
import functools

import jax
import jax.numpy as jnp
from jax import lax
from jax.experimental import pallas as pl
from jax.experimental.pallas import tpu as pltpu
from jax.experimental.pallas import tpu_sc as plsc

GRID_RES = 128
GRID_DIM = 8
NUM_EMBEDDINGS = 12000
EMBEDDING_DIM = 8
TABLE_SIZE_ = NUM_EMBEDDINGS * GRID_DIM ** 3
RES_ = GRID_RES * GRID_DIM
HI_ = float(RES_ - 1 - 1e-5)
SCALE_ = 0.5 * (RES_ - 1)
K2_ = 2654435761
K3_ = 805459861

NC = 2
NS = 16
NW = NC * NS
L = 16

C = 512
NR = 8 * C
NSTREAM = NR // 128

CORNERS = [(dx, dy, dz) for dx in (0, 1) for dy in (0, 1) for dz in (0, 1)]

Bt = 4096


K2S_ = K2_ - 2 ** 32
K3S_ = K3_
TWO32_ = float(2 ** 32)
RCP_T_ = 1.0 / TABLE_SIZE_


def _vfull(val, dtype=jnp.float32):
    return jnp.full((16,), val, dtype)


def _dims(v):
    p_raw = (v + 1.0) * 0.5 * float(RES_ - 1)
    p = jnp.minimum(jnp.maximum(p_raw, 0.0), jnp.float32(HI_))
    i0 = p.astype(jnp.int32)
    fr = p - i0.astype(jnp.float32)
    i1 = jnp.minimum(i0 + 1, RES_ - 1)
    s = jnp.where((p_raw >= 0.0) & (p_raw <= jnp.float32(HI_)),
                  _vfull(SCALE_), _vfull(0.0))
    return i0, i1, fr, s


def _hash_mod(hx, hy, hz):
    h = hx ^ (hy * jnp.int32(K2S_)) ^ (hz * jnp.int32(K3S_))
    uh = h.astype(jnp.float32) + jnp.where(h < 0, _vfull(TWO32_), _vfull(0.0))
    q = (uh * jnp.float32(RCP_T_)).astype(jnp.int32)
    r = h - q * jnp.int32(TABLE_SIZE_)
    r = r + jnp.where(r < 0, _vfull(TABLE_SIZE_, jnp.int32),
                      _vfull(0, jnp.int32))
    r = r - jnp.where(r >= TABLE_SIZE_, _vfull(TABLE_SIZE_, jnp.int32),
                      _vfull(0, jnp.int32))
    return r


BPT = (TABLE_SIZE_ // 128) // NW
NBLK = 15


def _sc_transpose_body(tin, tout, in_v, out_v):
    cid = lax.axis_index("c")
    sid = lax.axis_index("s")
    wid = sid * NC + cid
    lanes = lax.iota(jnp.int32, 16)

    @pl.loop(0, BPT // NBLK)
    def _ch(k):
        b0 = wid * BPT + k * NBLK
        pltpu.sync_copy(tin.at[pl.ds(b0 * 1024, NBLK * 1024)], in_v)

        @pl.loop(0, NBLK * 8)
        def _g(g):
            eb = g // 8
            o = (g % 8) * 16
            for d in range(8):
                t = in_v[pl.ds(eb * 1024 + d * 128 + o, 16)]
                plsc.store_scatter(out_v, [(eb * 128 + o + lanes) * 8 + d], t)

        pltpu.sync_copy(out_v, tout.at[pl.ds(b0 * 1024, NBLK * 1024)])


def _sc_body(x_hbm, y_hbm, z_hbm, table_hbm, feats_hbm, mask_hbm,
             pos_v, idx_v, rows_v, out_v, mask_v, gsem, *, k_chunks, n_pad):
    cid = lax.axis_index("c")
    sid = lax.axis_index("s")
    wid = sid * NC + cid
    lanes = lax.iota(jnp.int32, 16)

    @pl.loop(0, k_chunks)
    def _chunk(k):
        base = (wid * k_chunks + k) * C
        pltpu.sync_copy(x_hbm.at[pl.ds(base, C)], pos_v.at[pl.ds(0, C)])
        pltpu.sync_copy(y_hbm.at[pl.ds(base, C)], pos_v.at[pl.ds(C, C)])
        pltpu.sync_copy(z_hbm.at[pl.ds(base, C)], pos_v.at[pl.ds(2 * C, C)])

        def _xyz(o):
            x = pos_v[pl.ds(o, 16)]
            y = pos_v[pl.ds(C + o, 16)]
            z = pos_v[pl.ds(2 * C + o, 16)]
            return x, y, z

        @pl.loop(0, C // 16)
        def _grp1(g):
            o = g * 16
            x, y, z = _xyz(o)
            ix0, ix1, _, _ = _dims(x)
            iy0, iy1, _, _ = _dims(y)
            iz0, iz1, _, _ = _dims(z)
            for c, (dx, dy, dz) in enumerate(CORNERS):
                h = _hash_mod((ix0, ix1)[dx], (iy0, iy1)[dy], (iz0, iz1)[dz])
                idx_v[pl.ds(c * C + o, 16)] = h
            m = ((jnp.abs(x) <= 1.0) & (jnp.abs(y) <= 1.0)
                 & (jnp.abs(z) <= 1.0))
            mask_v[pl.ds(o, 16)] = jnp.where(
                m, _vfull(1, jnp.int32), _vfull(0, jnp.int32))

        pltpu.async_copy(table_hbm.at[idx_v], rows_v, gsem).wait()

        @pl.loop(0, C // 16)
        def _grp2(g):
            o = g * 16
            x, y, z = _xyz(o)
            _, _, fx_, sx = _dims(x)
            _, _, fy_, sy = _dims(y)
            _, _, fz_, sz = _dims(z)
            wx = (1.0 - fx_, fx_)
            wy = (1.0 - fy_, fy_)
            wz = (1.0 - fz_, fz_)
            wyz = {(a, b): wy[a] * wz[b] for a in (0, 1) for b in (0, 1)}
            wxz = {(a, b): wx[a] * wz[b] for a in (0, 1) for b in (0, 1)}
            wxy = {(a, b): wx[a] * wy[b] for a in (0, 1) for b in (0, 1)}
            uyz = {kk: v * sx for kk, v in wyz.items()}
            uxz = {kk: v * sy for kk, v in wxz.items()}
            uxy = {kk: v * sz for kk, v in wxy.items()}
            zero = jnp.zeros((16,), jnp.float32)
            accF = [zero] * 8
            accX = [zero] * 8
            accY = [zero] * 8
            accZ = [zero] * 8
            for c, (dx, dy, dz) in enumerate(CORNERS):
                m_c = wx[dx] * wyz[(dy, dz)]
                ax = uyz[(dy, dz)]
                ay = uxz[(dx, dz)]
                az = uxy[(dx, dy)]
                rbase = c * C + o + lanes
                for d in range(8):
                    col = jnp.full((16,), d, jnp.int32)
                    t = plsc.load_gather(rows_v, [rbase, col])
                    accF[d] = accF[d] + m_c * t
                    accX[d] = accX[d] + ax * t if dx else accX[d] - ax * t
                    accY[d] = accY[d] + ay * t if dy else accY[d] - ay * t
                    accZ[d] = accZ[d] + az * t if dz else accZ[d] - az * t
            blk = (o // 128) * (32 * 128) + (o % 128)
            for d in range(8):
                out_v[pl.ds(blk + d * 128, 16)] = accF[d]
                out_v[pl.ds(blk + (8 + d) * 128, 16)] = accX[d]
                out_v[pl.ds(blk + (16 + d) * 128, 16)] = accY[d]
                out_v[pl.ds(blk + (24 + d) * 128, 16)] = accZ[d]

        pltpu.sync_copy(out_v, feats_hbm.at[pl.ds(base * 32, C * 32)])
        pltpu.sync_copy(mask_v, mask_hbm.at[pl.ds(base, C)])


def _tc_body(feats_ref, w0t_ref, w1t_ref, w1_ref, w0_ref, w2_ref, w2t_ref,
             bias_ref, out_ref):
    f32 = jnp.float32
    gb = feats_ref.shape[0]
    big = jnp.concatenate([feats_ref[g] for g in range(gb)], axis=1)
    F = big[0:8, :]
    FX = big[8:16, :]
    FY = big[16:24, :]
    FZ = big[24:32, :]
    b0 = bias_ref[:, 0:1]
    b1 = bias_ref[:, 1:2]
    b2 = bias_ref[0:1, 2:3]
    A0 = jnp.dot(w0t_ref[...], F, preferred_element_type=f32) + b0
    h0 = jnp.maximum(A0, 0.0)
    A1 = jnp.dot(w1t_ref[...], h0, preferred_element_type=f32) + b1
    h1 = jnp.maximum(A1, 0.0)
    sdf = jnp.dot(w2t_ref[...], h1, preferred_element_type=f32) + b2
    g1 = jnp.where(A1 > 0.0, w2_ref[...], 0.0)
    g0 = jnp.where(A0 > 0.0,
                   jnp.dot(w1_ref[...], g1, preferred_element_type=f32), 0.0)
    gF = jnp.dot(w0_ref[...], g0, preferred_element_type=f32)
    gx = jnp.sum(gF * FX, axis=0, keepdims=True)
    gy = jnp.sum(gF * FY, axis=0, keepdims=True)
    gz = jnp.sum(gF * FZ, axis=0, keepdims=True)
    out_ref[0:4, :] = jnp.concatenate([sdf, gx, gy, gz], axis=0)


def kernel(positions, table, W0, b0, W1, b1, W2, b2):
    n = positions.shape[0]
    span = NW * C
    k_chunks = -(-n // span)
    n_pad = k_chunks * span

    xp = jnp.pad(positions[:, 0], (0, n_pad - n))
    yp = jnp.pad(positions[:, 1], (0, n_pad - n))
    zp = jnp.pad(positions[:, 2], (0, n_pad - n))

    mesh = plsc.VectorSubcoreMesh(core_axis_name="c", subcore_axis_name="s",
                                  num_cores=NC, num_subcores=NS)
    sc_params = pltpu.CompilerParams(use_tc_tiling_on_sc=False,
                                     needs_layout_passes=False)

    tiled_flat = table.reshape(TABLE_SIZE_ // 128, 128, 8)
    tiled_flat = tiled_flat.swapaxes(1, 2).reshape(-1)
    tr_fn = pl.kernel(
        _sc_transpose_body,
        out_type=jax.ShapeDtypeStruct((TABLE_SIZE_ * 8,), jnp.float32),
        mesh=mesh,
        scratch_types=[
            pltpu.VMEM((NBLK * 1024,), jnp.float32),
            pltpu.VMEM((NBLK * 1024,), jnp.float32),
        ],
        compiler_params=sc_params,
    )
    table_lin = tr_fn(tiled_flat).reshape(TABLE_SIZE_, 8)

    sc_fn = pl.kernel(
        functools.partial(_sc_body, k_chunks=k_chunks, n_pad=n_pad),
        out_type=(jax.ShapeDtypeStruct((32 * n_pad,), jnp.float32),
                  jax.ShapeDtypeStruct((n_pad,), jnp.int32)),
        mesh=mesh,
        scratch_types=[
            pltpu.VMEM((3 * C,), jnp.float32),
            pltpu.VMEM((NR,), jnp.int32),
            pltpu.VMEM((NR, 8), jnp.float32),
            pltpu.VMEM((32 * C,), jnp.float32),
            pltpu.VMEM((C,), jnp.int32),
            pltpu.SemaphoreType.DMA,
        ],
        compiler_params=sc_params,
    )
    feats, mask_i = sc_fn(xp, yp, zp, table_lin)

    bias_pack = jnp.zeros((32, 128), jnp.float32)
    bias_pack = bias_pack.at[:, 0].set(b0)
    bias_pack = bias_pack.at[:, 1].set(b1)
    bias_pack = bias_pack.at[0, 2].set(b2[0])

    feats3 = feats.reshape(n_pad // 128, 32, 128)
    grid = n_pad // Bt
    out4 = pl.pallas_call(
        _tc_body,
        grid=(grid,),
        in_specs=[
            pl.BlockSpec((Bt // 128, 32, 128), lambda i: (i, 0, 0)),
            pl.BlockSpec((32, 8), lambda i: (0, 0)),
            pl.BlockSpec((32, 32), lambda i: (0, 0)),
            pl.BlockSpec((32, 32), lambda i: (0, 0)),
            pl.BlockSpec((8, 32), lambda i: (0, 0)),
            pl.BlockSpec((32, 1), lambda i: (0, 0)),
            pl.BlockSpec((1, 32), lambda i: (0, 0)),
            pl.BlockSpec((32, 128), lambda i: (0, 0)),
        ],
        out_specs=pl.BlockSpec((8, Bt), lambda i: (0, i)),
        out_shape=jax.ShapeDtypeStruct((8, n_pad), jnp.float32),
    )(feats3, W0.T, W1.T, W1, W0, W2, W2.T, bias_pack)

    sdf = out4[0, :n].reshape(n, 1)
    grad_x = out4[1:4, :n].T
    mask = mask_i[:n].astype(bool)
    return (sdf, grad_x, mask)
